# baseline (speedup 1.0000x reference)
import functools
import math

import jax
import jax.numpy as jnp
from jax import lax
from jax.experimental import pallas as pl
from jax.experimental.pallas import tpu as pltpu
from jax.experimental.pallas import tpu_sc as plsc

L = 2048
GEO = 1536
SEM = 512
RSA = 64
OUT = 256
H = 8
DH = 32
K = 10
BLK = 256
GRID = L // BLK

NC = 2
NS = 16
NW = NC * NS
LANES = 16
RPW = L // NW
NG = RPW // LANES


def _ln(x, g, b):
    m = jnp.mean(x, axis=-1, keepdims=True)
    v = jnp.mean((x - m) ** 2, axis=-1, keepdims=True)
    return (x - m) / jnp.sqrt(v + 1e-5) * g + b


def _leaky(x):
    return jnp.where(x >= 0, x, 0.01 * x)


def _projqkv_body(geo_ref, sem_ref, rsa_ref,
                  geo_w, geo_b, geo_g, geo_bb,
                  sem_w, sem_b, sem_g, sem_bb,
                  rsa_w, rsa_b, rsa_g, rsa_bb,
                  rt_w, rt_b, rt_g, rt_bb,
                  wq, bq, wkv, bkv,
                  geo_p_o, sem_p_o, rsa_o_o,
                  q_geo_o, kv_geo_o, q_sem_o, kv_sem_o):
    f32 = jnp.float32
    geo_p = _ln(jnp.dot(geo_ref[...], geo_w[...], preferred_element_type=f32)
                + geo_b[...], geo_g[...], geo_bb[...])
    sem_p = _ln(jnp.dot(sem_ref[...], sem_w[...], preferred_element_type=f32)
                + sem_b[...], sem_g[...], sem_bb[...])
    rsa_p = _ln(jnp.dot(rsa_ref[...], rsa_w[...], preferred_element_type=f32)
                + rsa_b[...], rsa_g[...], rsa_bb[...])
    rsa_o_o[...] = _leaky(_ln(
        jnp.dot(rsa_p, rt_w[...], preferred_element_type=f32) + rt_b[...],
        rt_g[...], rt_bb[...]))
    geo_p_o[...] = geo_p
    sem_p_o[...] = sem_p
    q_geo_o[...] = jnp.dot(geo_p, wq[...], preferred_element_type=f32) + bq[...]
    q_sem_o[...] = jnp.dot(sem_p, wq[...], preferred_element_type=f32) + bq[...]
    kv_geo_o[...] = jnp.dot(geo_p, wkv[...], preferred_element_type=f32) + bkv[...]
    kv_sem_o[...] = jnp.dot(sem_p, wkv[...], preferred_element_type=f32) + bkv[...]


def _topk_body(pos_ref, pos_t_ref, idx_o):
    i = pl.program_id(0)
    rows = pl.ds(i * BLK, BLK)

    x_r = pos_ref[rows, 0:1]
    y_r = pos_ref[rows, 1:2]
    z_r = pos_ref[rows, 2:3]
    x_c = pos_t_ref[0:1, :]
    y_c = pos_t_ref[1:2, :]
    z_c = pos_t_ref[2:3, :]
    sq_r = x_r * x_r + y_r * y_r + z_r * z_r
    sq_c = x_c * x_c + y_c * y_c + z_c * z_c
    dot_rc = x_r * x_c + y_r * y_c + z_r * z_c
    d2 = sq_r + sq_c - 2.0 * dot_rc
    d = jnp.sqrt(jnp.maximum(d2, 0.0))

    colid = jax.lax.broadcasted_iota(jnp.int32, (BLK, L), 1)
    idxs = []
    for _ in range(K):
        m = jnp.min(d, axis=1, keepdims=True)
        cand = jnp.where(d == m, colid, jnp.int32(L))
        imin = jnp.min(cand, axis=1, keepdims=True)
        sel = colid == imin
        idxs.append(imin)
        d = jnp.where(sel, jnp.inf, d)
    idx_o[...] = jnp.concatenate(idxs, axis=1)


def _sc_attn_body(idx_hbm, qg_hbm, kvs_hbm, qs_hbm, kvg_hbm,
                  og_hbm, os_hbm,
                  idx_v, q_v, kv_v, out_v, sem):
    wid = lax.axis_index("s") * NC + lax.axis_index("c")
    base = wid * RPW
    pltpu.sync_copy(idx_hbm.at[pl.ds(base * K, RPW * K)], idx_v)

    iota = lax.iota(jnp.int32, LANES)
    row_idx = iota * K
    scale = jnp.float32(1.0 / math.sqrt(DH))

    def one_dir(g, q_hbm, kv_hbm, out_hbm):
        row0 = base + g * LANES
        pltpu.sync_copy(q_hbm.at[pl.ds(row0, LANES)], q_v)
        cp0 = pltpu.async_copy(
            kv_hbm.at[idx_v.at[pl.ds(g * (LANES * K), 80)]],
            kv_v.at[pl.ds(0, 80)], sem)
        cp1 = pltpu.async_copy(
            kv_hbm.at[idx_v.at[pl.ds(g * (LANES * K) + 80, 80)]],
            kv_v.at[pl.ds(80, 80)], sem)
        cp0.wait()
        cp1.wait()

        def head(h, carry):
            c0 = h * DH
            accs = [jnp.zeros((LANES,), jnp.float32) for _ in range(K)]
            for dd in range(DH):
                colv = iota * 0 + (c0 + dd)
                qv = plsc.load_gather(q_v, [iota, colv])
                for j in range(K):
                    kvj = plsc.load_gather(kv_v, [row_idx + j, colv])
                    accs[j] = accs[j] + qv * kvj
            svecs = [a * scale for a in accs]
            m = svecs[0]
            for j in range(1, K):
                m = jnp.maximum(m, svecs[j])
            ps = [jnp.exp(s - m) for s in svecs]
            den = ps[0]
            for j in range(1, K):
                den = den + ps[j]
            inv = jnp.float32(1.0) / den
            avs = [p * inv for p in ps]
            for dd in range(DH):
                colv = iota * 0 + (c0 + dd)
                vcolv = colv + OUT
                acc = jnp.zeros((LANES,), jnp.float32)
                for j in range(K):
                    vvj = plsc.load_gather(kv_v, [row_idx + j, vcolv])
                    acc = acc + avs[j] * vvj
                plsc.store_scatter(out_v, [iota, colv], acc)
            return carry

        lax.fori_loop(0, H, head, 0)
        pltpu.sync_copy(out_v, out_hbm.at[pl.ds(row0, LANES)])

    def group(g, carry):
        one_dir(g, qg_hbm, kvs_hbm, og_hbm)
        one_dir(g, qs_hbm, kvg_hbm, os_hbm)
        return carry

    lax.fori_loop(0, NG, group, 0)


_sc_attn = functools.partial(
    pl.kernel,
    out_type=[jax.ShapeDtypeStruct((L, OUT), jnp.float32),
              jax.ShapeDtypeStruct((L, OUT), jnp.float32)],
    mesh=plsc.VectorSubcoreMesh(core_axis_name="c", subcore_axis_name="s",
                                num_cores=NC, num_subcores=NS),
    compiler_params=pltpu.CompilerParams(use_tc_tiling_on_sc=False,
                                         needs_layout_passes=False),
    scratch_types=[
        pltpu.VMEM((RPW * K,), jnp.int32),
        pltpu.VMEM((LANES, OUT), jnp.float32),
        pltpu.VMEM((LANES * K, 2 * OUT), jnp.float32),
        pltpu.VMEM((LANES, OUT), jnp.float32),
        pltpu.SemaphoreType.DMA,
    ],
)(_sc_attn_body)


def _head_body(geo_p_ref, sem_p_ref, rsa_o_ref, attn_geo_ref, attn_sem_ref,
               o_w, o_b, ab_ref, ln1_g, ln1_b, ln2_g, ln2_b,
               f1_w, f1_b, f1_g, f1_bb, f2_w, f2_b, f2_g, f2_bb,
               out_o):
    f32 = jnp.float32
    a_geo = ab_ref[0, 0]
    b_geo = ab_ref[0, 1]
    a_sem = ab_ref[0, 2]
    b_sem = ab_ref[0, 3]
    ag = jnp.dot(attn_geo_ref[...], o_w[...], preferred_element_type=f32) + o_b[...]
    geo_out = _ln(a_geo * geo_p_ref[...] + b_geo * ag, ln1_g[...], ln1_b[...])
    asm = jnp.dot(attn_sem_ref[...], o_w[...], preferred_element_type=f32) + o_b[...]
    sem_out = _ln(a_sem * sem_p_ref[...] + b_sem * asm, ln2_g[...], ln2_b[...])
    h = (jnp.dot(geo_out, f1_w[0:OUT, :], preferred_element_type=f32)
         + jnp.dot(sem_out, f1_w[OUT:2 * OUT, :], preferred_element_type=f32)
         + jnp.dot(rsa_o_ref[...], f1_w[2 * OUT:3 * OUT, :], preferred_element_type=f32)
         + f1_b[...])
    h = _leaky(_ln(h, f1_g[...], f1_bb[...]))
    f = _leaky(_ln(jnp.dot(h, f2_w[...], preferred_element_type=f32) + f2_b[...],
                   f2_g[...], f2_bb[...]))
    out_o[...] = f


def _row_spec(d):
    return pl.BlockSpec((BLK, d), lambda i: (i, 0))


def _full_spec(shape):
    n = len(shape)
    return pl.BlockSpec(shape, lambda i: (0,) * n)


@jax.jit
def kernel(geo_feat, sem_feat, rsa_feat, pos, params):
    p = params
    f32 = jnp.float32
    row = lambda v: jnp.reshape(v, (1, -1)).astype(f32)

    wkv = jnp.concatenate([p['k_w'], p['v_w']], axis=1)
    bkv = jnp.concatenate([p['k_b'], p['v_b']]).reshape(1, -1)

    (geo_p, sem_p, rsa_out,
     q_geo, kv_geo, q_sem, kv_sem) = pl.pallas_call(
        _projqkv_body,
        grid=(GRID,),
        in_specs=[
            _row_spec(GEO), _row_spec(SEM), _row_spec(RSA),
            _full_spec((GEO, OUT)), _full_spec((1, OUT)), _full_spec((1, OUT)), _full_spec((1, OUT)),
            _full_spec((SEM, OUT)), _full_spec((1, OUT)), _full_spec((1, OUT)), _full_spec((1, OUT)),
            _full_spec((RSA, OUT)), _full_spec((1, OUT)), _full_spec((1, OUT)), _full_spec((1, OUT)),
            _full_spec((OUT, OUT)), _full_spec((1, OUT)), _full_spec((1, OUT)), _full_spec((1, OUT)),
            _full_spec((OUT, OUT)), _full_spec((1, OUT)),
            _full_spec((OUT, 2 * OUT)), _full_spec((1, 2 * OUT)),
        ],
        out_specs=[_row_spec(OUT), _row_spec(OUT), _row_spec(OUT),
                   _row_spec(OUT), _row_spec(2 * OUT),
                   _row_spec(OUT), _row_spec(2 * OUT)],
        out_shape=[
            jax.ShapeDtypeStruct((L, OUT), f32),
            jax.ShapeDtypeStruct((L, OUT), f32),
            jax.ShapeDtypeStruct((L, OUT), f32),
            jax.ShapeDtypeStruct((L, OUT), f32),
            jax.ShapeDtypeStruct((L, 2 * OUT), f32),
            jax.ShapeDtypeStruct((L, OUT), f32),
            jax.ShapeDtypeStruct((L, 2 * OUT), f32),
        ],
    )(geo_feat, sem_feat, rsa_feat,
      p['geo_w'], row(p['geo_b']), row(p['geo_g']), row(p['geo_bb']),
      p['sem_w'], row(p['sem_b']), row(p['sem_g']), row(p['sem_bb']),
      p['rsa_w'], row(p['rsa_b']), row(p['rsa_g']), row(p['rsa_bb']),
      p['rt_w'], row(p['rt_b']), row(p['rt_g']), row(p['rt_bb']),
      p['q_w'], row(p['q_b']), wkv, bkv)

    pos_t = pos.T
    idx = pl.pallas_call(
        _topk_body,
        grid=(GRID,),
        in_specs=[_full_spec((L, 3)), _full_spec((3, L))],
        out_specs=_row_spec(K),
        out_shape=jax.ShapeDtypeStruct((L, K), jnp.int32),
    )(pos, pos_t)

    attn_geo, attn_sem = _sc_attn(
        idx.reshape(-1), q_geo, kv_sem, q_sem, kv_geo)

    ab = jnp.stack([p['a_geo'], p['b_geo'], p['a_sem'], p['b_sem']]).reshape(1, 4)
    fused = pl.pallas_call(
        _head_body,
        grid=(GRID,),
        in_specs=[_row_spec(OUT), _row_spec(OUT), _row_spec(OUT),
                  _row_spec(OUT), _row_spec(OUT),
                  _full_spec((OUT, OUT)), _full_spec((1, OUT)),
                  _full_spec((1, 4)),
                  _full_spec((1, OUT)), _full_spec((1, OUT)),
                  _full_spec((1, OUT)), _full_spec((1, OUT)),
                  _full_spec((3 * OUT, 2 * OUT)), _full_spec((1, 2 * OUT)),
                  _full_spec((1, 2 * OUT)), _full_spec((1, 2 * OUT)),
                  _full_spec((2 * OUT, OUT)), _full_spec((1, OUT)),
                  _full_spec((1, OUT)), _full_spec((1, OUT))],
        out_specs=_row_spec(OUT),
        out_shape=jax.ShapeDtypeStruct((L, OUT), f32),
    )(geo_p, sem_p, rsa_out, attn_geo, attn_sem,
      p['o_w'], row(p['o_b']), ab,
      row(p['ln1_g']), row(p['ln1_b']), row(p['ln2_g']), row(p['ln2_b']),
      p['f1_w'], row(p['f1_b']), row(p['f1_g']), row(p['f1_bb']),
      p['f2_w'], row(p['f2_b']), row(p['f2_g']), row(p['f2_bb']))
    return fused


# speedup vs baseline: 2.0211x; 2.0211x over previous
import functools
import math

import jax
import jax.numpy as jnp
from jax import lax
from jax.experimental import pallas as pl
from jax.experimental.pallas import tpu as pltpu
from jax.experimental.pallas import tpu_sc as plsc

L = 2048
GEO = 1536
SEM = 512
RSA = 64
OUT = 256
H = 8
DH = 32
K = 10
BLK = 256
GRID = L // BLK

NC = 2
NS = 16
NW = NC * NS
RPW = L // NW
SPW = RPW * K
CH = 80
NCH = SPW // CH


def _ln(x, g, b):
    m = jnp.mean(x, axis=-1, keepdims=True)
    v = jnp.mean((x - m) ** 2, axis=-1, keepdims=True)
    return (x - m) * jax.lax.rsqrt(v + 1e-5) * g + b


def _leaky(x):
    return jnp.where(x >= 0, x, 0.01 * x)


def _projqkv_body(geo_ref, sem_ref, rsa_ref,
                  geo_w, geo_b, geo_g, geo_bb,
                  sem_w, sem_b, sem_g, sem_bb,
                  rsa_w, rsa_b, rsa_g, rsa_bb,
                  rt_w, rt_b, rt_g, rt_bb,
                  wq, bq, wkv, bkv,
                  geo_p_o, sem_p_o, rsa_o_o,
                  q_geo_o, kv_geo_o, q_sem_o, kv_sem_o):
    f32 = jnp.float32
    geo_p = _ln(jnp.dot(geo_ref[...], geo_w[...], preferred_element_type=f32)
                + geo_b[...], geo_g[...], geo_bb[...])
    sem_p = _ln(jnp.dot(sem_ref[...], sem_w[...], preferred_element_type=f32)
                + sem_b[...], sem_g[...], sem_bb[...])
    rsa_p = _ln(jnp.dot(rsa_ref[...], rsa_w[...], preferred_element_type=f32)
                + rsa_b[...], rsa_g[...], rsa_bb[...])
    rsa_o_o[...] = _leaky(_ln(
        jnp.dot(rsa_p, rt_w[...], preferred_element_type=f32) + rt_b[...],
        rt_g[...], rt_bb[...]))
    geo_p_o[...] = geo_p
    sem_p_o[...] = sem_p
    q_geo_o[...] = jnp.dot(geo_p, wq[...], preferred_element_type=f32) + bq[...]
    q_sem_o[...] = jnp.dot(sem_p, wq[...], preferred_element_type=f32) + bq[...]
    kv_geo_o[...] = jnp.dot(geo_p, wkv[...], preferred_element_type=f32) + bkv[...]
    kv_sem_o[...] = jnp.dot(sem_p, wkv[...], preferred_element_type=f32) + bkv[...]


def _topk_body(pos_ref, pos_t_ref, idx_o):
    i = pl.program_id(0)
    rows = pl.ds(i * BLK, BLK)

    x_r = pos_ref[rows, 0:1]
    y_r = pos_ref[rows, 1:2]
    z_r = pos_ref[rows, 2:3]
    x_c = pos_t_ref[0:1, :]
    y_c = pos_t_ref[1:2, :]
    z_c = pos_t_ref[2:3, :]
    sq_r = x_r * x_r + y_r * y_r + z_r * z_r
    sq_c = x_c * x_c + y_c * y_c + z_c * z_c
    dot_rc = x_r * x_c + y_r * y_c + z_r * z_c
    d2 = sq_r + sq_c - 2.0 * dot_rc
    d = jnp.sqrt(jnp.maximum(d2, 0.0))

    colid = jax.lax.broadcasted_iota(jnp.int32, (BLK, L), 1).astype(jnp.float32)
    idxs = []
    for _ in range(K):
        m = jnp.min(d, axis=1, keepdims=True)
        cand = jnp.where(d == m, colid, jnp.float32(3e8))
        imin = jnp.min(cand, axis=1, keepdims=True)
        sel = colid == imin
        idxs.append(imin)
        d = jnp.where(sel, jnp.inf, d)
    idx_o[...] = jnp.concatenate(idxs, axis=1).astype(jnp.int32)


def _sc_gather_body(idxt_hbm, kvs_hbm, kvg_hbm,
                    kvcg_hbm, kvcs_hbm,
                    idx_v, buf0, buf1, sem_g, sem_s):
    wid = lax.axis_index("s") * NC + lax.axis_index("c")
    base = wid * SPW
    pltpu.sync_copy(idxt_hbm.at[pl.ds(base, SPW)], idx_v)

    def do_dir(kv_hbm, out_hbm):
        def chunk2(c2, carry):
            c = c2 * 2
            cpa = pltpu.async_copy(
                kv_hbm.at[idx_v.at[pl.ds(c * CH, CH)]], buf0, sem_g)
            cpa.wait()
            sta = pltpu.async_copy(
                buf0, out_hbm.at[pl.ds(base + c * CH, CH)], sem_s)
            cpb = pltpu.async_copy(
                kv_hbm.at[idx_v.at[pl.ds((c + 1) * CH, CH)]], buf1, sem_g)
            cpb.wait()
            sta.wait()
            stb = pltpu.async_copy(
                buf1, out_hbm.at[pl.ds(base + (c + 1) * CH, CH)], sem_s)
            stb.wait()
            return carry
        lax.fori_loop(0, NCH // 2, chunk2, 0)

    do_dir(kvs_hbm, kvcg_hbm)
    do_dir(kvg_hbm, kvcs_hbm)


_sc_gather = functools.partial(
    pl.kernel,
    out_type=[jax.ShapeDtypeStruct((L * K, 2 * OUT), jnp.float32),
              jax.ShapeDtypeStruct((L * K, 2 * OUT), jnp.float32)],
    mesh=plsc.VectorSubcoreMesh(core_axis_name="c", subcore_axis_name="s",
                                num_cores=NC, num_subcores=NS),
    compiler_params=pltpu.CompilerParams(use_tc_tiling_on_sc=False,
                                         needs_layout_passes=False),
    scratch_types=[
        pltpu.VMEM((SPW,), jnp.int32),
        pltpu.VMEM((CH, 2 * OUT), jnp.float32),
        pltpu.VMEM((CH, 2 * OUT), jnp.float32),
        pltpu.SemaphoreType.DMA,
        pltpu.SemaphoreType.DMA,
    ],
)(_sc_gather_body)


def _attend_compact(q, kvc_ref, segh, expand):
    f32 = jnp.float32
    scale = jnp.float32(1.0 / math.sqrt(DH))
    qs = q * scale
    svals = []
    for j in range(K):
        pr = qs * kvc_ref[j, :, 0:OUT]
        svals.append(jnp.dot(pr, segh, preferred_element_type=f32))
    m = svals[0]
    for j in range(1, K):
        m = jnp.maximum(m, svals[j])
    ps = [jnp.exp(s - m) for s in svals]
    den = ps[0]
    for j in range(1, K):
        den = den + ps[j]
    inv = 1.0 / den
    a_small = jnp.concatenate([p * inv for p in ps], axis=1)
    a_full = jnp.dot(a_small, expand, preferred_element_type=f32)
    out = a_full[:, 0:OUT] * kvc_ref[0, :, OUT:2 * OUT]
    for j in range(1, K):
        out = out + a_full[:, j * OUT:(j + 1) * OUT] * kvc_ref[j, :, OUT:2 * OUT]
    return out


def _attn_head_body(q_geo_ref, q_sem_ref, kvcg_ref, kvcs_ref,
                    geo_p_ref, sem_p_ref, rsa_o_ref, segh_ref, expand_ref,
                    o_w, o_b, ab_ref, ln1_g, ln1_b, ln2_g, ln2_b,
                    f1_w, f1_b, f1_g, f1_bb, f2_w, f2_b, f2_g, f2_bb,
                    out_o):
    f32 = jnp.float32
    segh = segh_ref[...]
    expand = expand_ref[...]
    attn_geo = _attend_compact(q_geo_ref[...], kvcg_ref, segh, expand)
    attn_sem = _attend_compact(q_sem_ref[...], kvcs_ref, segh, expand)
    a_geo = ab_ref[0, 0]
    b_geo = ab_ref[0, 1]
    a_sem = ab_ref[0, 2]
    b_sem = ab_ref[0, 3]
    ag = jnp.dot(attn_geo, o_w[...], preferred_element_type=f32) + o_b[...]
    geo_out = _ln(a_geo * geo_p_ref[...] + b_geo * ag, ln1_g[...], ln1_b[...])
    asm = jnp.dot(attn_sem, o_w[...], preferred_element_type=f32) + o_b[...]
    sem_out = _ln(a_sem * sem_p_ref[...] + b_sem * asm, ln2_g[...], ln2_b[...])
    h = (jnp.dot(geo_out, f1_w[0:OUT, :], preferred_element_type=f32)
         + jnp.dot(sem_out, f1_w[OUT:2 * OUT, :], preferred_element_type=f32)
         + jnp.dot(rsa_o_ref[...], f1_w[2 * OUT:3 * OUT, :], preferred_element_type=f32)
         + f1_b[...])
    h = _leaky(_ln(h, f1_g[...], f1_bb[...]))
    f = _leaky(_ln(jnp.dot(h, f2_w[...], preferred_element_type=f32) + f2_b[...],
                   f2_g[...], f2_bb[...]))
    out_o[...] = f


def _row_spec(d):
    return pl.BlockSpec((BLK, d), lambda i: (i, 0))


def _full_spec(shape):
    n = len(shape)
    return pl.BlockSpec(shape, lambda i: (0,) * n)


@jax.jit
def kernel(geo_feat, sem_feat, rsa_feat, pos, params):
    p = params
    f32 = jnp.float32
    row = lambda v: jnp.reshape(v, (1, -1)).astype(f32)

    wkv = jnp.concatenate([p['k_w'], p['v_w']], axis=1)
    bkv = jnp.concatenate([p['k_b'], p['v_b']]).reshape(1, -1)

    (geo_p, sem_p, rsa_out,
     q_geo, kv_geo, q_sem, kv_sem) = pl.pallas_call(
        _projqkv_body,
        grid=(GRID,),
        in_specs=[
            _row_spec(GEO), _row_spec(SEM), _row_spec(RSA),
            _full_spec((GEO, OUT)), _full_spec((1, OUT)), _full_spec((1, OUT)), _full_spec((1, OUT)),
            _full_spec((SEM, OUT)), _full_spec((1, OUT)), _full_spec((1, OUT)), _full_spec((1, OUT)),
            _full_spec((RSA, OUT)), _full_spec((1, OUT)), _full_spec((1, OUT)), _full_spec((1, OUT)),
            _full_spec((OUT, OUT)), _full_spec((1, OUT)), _full_spec((1, OUT)), _full_spec((1, OUT)),
            _full_spec((OUT, OUT)), _full_spec((1, OUT)),
            _full_spec((OUT, 2 * OUT)), _full_spec((1, 2 * OUT)),
        ],
        out_specs=[_row_spec(OUT), _row_spec(OUT), _row_spec(OUT),
                   _row_spec(OUT), _row_spec(2 * OUT),
                   _row_spec(OUT), _row_spec(2 * OUT)],
        out_shape=[
            jax.ShapeDtypeStruct((L, OUT), f32),
            jax.ShapeDtypeStruct((L, OUT), f32),
            jax.ShapeDtypeStruct((L, OUT), f32),
            jax.ShapeDtypeStruct((L, OUT), f32),
            jax.ShapeDtypeStruct((L, 2 * OUT), f32),
            jax.ShapeDtypeStruct((L, OUT), f32),
            jax.ShapeDtypeStruct((L, 2 * OUT), f32),
        ],
    )(geo_feat, sem_feat, rsa_feat,
      p['geo_w'], row(p['geo_b']), row(p['geo_g']), row(p['geo_bb']),
      p['sem_w'], row(p['sem_b']), row(p['sem_g']), row(p['sem_bb']),
      p['rsa_w'], row(p['rsa_b']), row(p['rsa_g']), row(p['rsa_bb']),
      p['rt_w'], row(p['rt_b']), row(p['rt_g']), row(p['rt_bb']),
      p['q_w'], row(p['q_b']), wkv, bkv)

    pos_t = pos.T
    idx = pl.pallas_call(
        _topk_body,
        grid=(GRID,),
        in_specs=[_full_spec((L, 3)), _full_spec((3, L))],
        out_specs=_row_spec(K),
        out_shape=jax.ShapeDtypeStruct((L, K), jnp.int32),
    )(pos, pos_t)

    idxt = idx.T.reshape(-1)
    kvc_geo, kvc_sem = _sc_gather(idxt, kv_sem, kv_geo)
    kvc_geo3 = kvc_geo.reshape(K, L, 2 * OUT)
    kvc_sem3 = kvc_sem.reshape(K, L, 2 * OUT)

    lanes = jnp.arange(OUT)
    segh = (lanes[:, None] // DH == jnp.arange(H)[None, :]).astype(f32)
    jh = jnp.arange(K * H)
    co = jnp.arange(K * OUT)
    expand = ((jh[:, None] // H == co[None, :] // OUT)
              & (jh[:, None] % H == (co[None, :] % OUT) // DH)).astype(f32)

    ab = jnp.stack([p['a_geo'], p['b_geo'], p['a_sem'], p['b_sem']]).reshape(1, 4)
    fused = pl.pallas_call(
        _attn_head_body,
        grid=(GRID,),
        in_specs=[_row_spec(OUT), _row_spec(OUT),
                  pl.BlockSpec((K, BLK, 2 * OUT), lambda i: (0, i, 0)),
                  pl.BlockSpec((K, BLK, 2 * OUT), lambda i: (0, i, 0)),
                  _row_spec(OUT), _row_spec(OUT), _row_spec(OUT),
                  _full_spec((OUT, H)), _full_spec((K * H, K * OUT)),
                  _full_spec((OUT, OUT)), _full_spec((1, OUT)),
                  _full_spec((1, 4)),
                  _full_spec((1, OUT)), _full_spec((1, OUT)),
                  _full_spec((1, OUT)), _full_spec((1, OUT)),
                  _full_spec((3 * OUT, 2 * OUT)), _full_spec((1, 2 * OUT)),
                  _full_spec((1, 2 * OUT)), _full_spec((1, 2 * OUT)),
                  _full_spec((2 * OUT, OUT)), _full_spec((1, OUT)),
                  _full_spec((1, OUT)), _full_spec((1, OUT))],
        out_specs=_row_spec(OUT),
        out_shape=jax.ShapeDtypeStruct((L, OUT), f32),
    )(q_geo, q_sem, kvc_geo3, kvc_sem3,
      geo_p, sem_p, rsa_out, segh, expand,
      p['o_w'], row(p['o_b']), ab,
      row(p['ln1_g']), row(p['ln1_b']), row(p['ln2_g']), row(p['ln2_b']),
      p['f1_w'], row(p['f1_b']), row(p['f1_g']), row(p['f1_bb']),
      p['f2_w'], row(p['f2_b']), row(p['f2_g']), row(p['f2_bb']))
    return fused
